# baseline (speedup 1.0000x reference)
import jax
import jax.numpy as jnp
from jax import lax
from jax.experimental import pallas as pl

B, N, NS = 8, 512, 16
C = 256
S = 8
CH = C // S
NCHUNK = 1
PC = N // NCHUNK
RC = PC * NS
CNT = float(B * N * NS)
EPS = 1e-5


def _onehot_rows(idxc):
    tgt = lax.broadcasted_iota(jnp.int32, (PC, NS, N), 2).astype(jnp.float32)
    sel = jnp.where(idxc[:, :, None] == tgt, 1.0, 0.0)
    return sel.reshape(RC, N)


def _rep_rows(a):
    d = a.shape[-1]
    return jnp.broadcast_to(a[:, None, :], (PC, NS, d)).reshape(RC, d)


def _bn_scale_shift(s_ref, ss_ref, gamma, beta):
    d = s_ref.shape[-1]
    ssum = jnp.sum(s_ref[...].reshape(B * NCHUNK, d), axis=0, keepdims=True)
    sssum = jnp.sum(ss_ref[...].reshape(B * NCHUNK, d), axis=0, keepdims=True)
    mean = ssum / CNT
    var = sssum / CNT - mean * mean
    scale = gamma / jnp.sqrt(var + EPS)
    shift = beta - mean * scale
    return scale, shift


def _pos_feat(pr0_ref, sp_ref, ssp_ref, gp_r, betap_r, wp1T_r, bp1_r):
    scalep, shiftp = _bn_scale_shift(sp_ref, ssp_ref, gp_r[...], betap_r[...])
    prn = jnp.maximum(pr0_ref[0] * scalep.reshape(3, 1) + shiftp.reshape(3, 1), 0.0)
    return lax.dot_general(prn, wp1T_r[...], (((0,), (0,)), ((), ())),
                           preferred_element_type=jnp.float32) + bp1_r[...]


def _k1_body(xt_ref, p_ref, pT_ref, wqT, bq, wkT, bk, wvT, bv, wp0T, bp0,
             q_ref, k_ref, v_ref, idx_ref, pr0_ref, sp_ref, ssp_ref):
    xb = xt_ref[0]
    dgq = lambda w: lax.dot_general(xb, w[...], (((0,), (0,)), ((), ())),
                                    preferred_element_type=jnp.float32)
    q_ref[0] = dgq(wqT) + bq[...]
    k_ref[0] = dgq(wkT) + bk[...]
    v_ref[0] = dgq(wvT) + bv[...]

    pc = p_ref[0]
    pT = pT_ref[0]
    dx = pc[:, 0:1] - pT[0:1, :]
    dy = pc[:, 1:2] - pT[1:2, :]
    dz = pc[:, 2:3] - pT[2:3, :]
    d2 = (dx * dx + dy * dy) + dz * dz

    colid = lax.broadcasted_iota(jnp.int32, (PC, N), 1).astype(jnp.float32)
    work = d2
    cols = []
    for _ in range(NS):
        m = jnp.min(work, axis=1, keepdims=True)
        cand = jnp.where(work == m, colid, jnp.float32(N))
        am = jnp.min(cand, axis=1, keepdims=True)
        cols.append(am)
        work = jnp.where(colid == am, jnp.inf, work)
    idxc = jnp.concatenate(cols, axis=1)
    idx_ref[0] = idxc

    sel = _onehot_rows(idxc)
    gp3 = lax.dot_general(sel, pT, (((1,), (1,)), ((), ())),
                          preferred_element_type=jnp.float32)
    prel = gp3 - _rep_rows(pc)
    pr0T = lax.dot_general(wp0T[...], prel, (((0,), (1,)), ((), ())),
                           preferred_element_type=jnp.float32) + bp0[...]
    pr0_ref[0] = pr0T
    sp_ref[0, 0] = jnp.sum(pr0T, axis=1, keepdims=True).reshape(1, 3)
    ssp_ref[0, 0] = jnp.sum(pr0T * pr0T, axis=1, keepdims=True).reshape(1, 3)


def _w0(q_ref, k_ref, idx_ref, p_r):
    sel = _onehot_rows(idx_ref[0])
    g_k = jnp.dot(sel, k_ref[0], preferred_element_type=jnp.float32)
    return g_k - _rep_rows(q_ref[0]) + p_r


def _k2_body(q_ref, k_ref, pr0_ref, idx_ref, sp_ref, ssp_ref,
             gp_r, betap_r, wp1T_r, bp1_r, s0_ref, ss0_ref):
    p_r = _pos_feat(pr0_ref, sp_ref, ssp_ref, gp_r, betap_r, wp1T_r, bp1_r)
    w0 = _w0(q_ref, k_ref, idx_ref, p_r)
    s0_ref[0, 0] = jnp.sum(w0, axis=0, keepdims=True)
    ss0_ref[0, 0] = jnp.sum(w0 * w0, axis=0, keepdims=True)


def _k3_body(q_ref, k_ref, pr0_ref, idx_ref, sp_ref, ssp_ref,
             gp_r, betap_r, wp1T_r, bp1_r,
             s0_ref, ss0_ref, gw0_r, bw0_r, ww0T_r, bw0l_r,
             h_ref, s1_ref, ss1_ref):
    p_r = _pos_feat(pr0_ref, sp_ref, ssp_ref, gp_r, betap_r, wp1T_r, bp1_r)
    w0 = _w0(q_ref, k_ref, idx_ref, p_r)
    scale0, shift0 = _bn_scale_shift(s0_ref, ss0_ref, gw0_r[...], bw0_r[...])
    w0n = jnp.maximum(w0 * scale0 + shift0, 0.0)
    hT = lax.dot_general(ww0T_r[...], w0n, (((0,), (1,)), ((), ())),
                         preferred_element_type=jnp.float32) + bw0l_r[...]
    h_ref[0] = hT
    s1_ref[0, 0] = jnp.sum(hT, axis=1, keepdims=True).reshape(1, CH)
    ss1_ref[0, 0] = jnp.sum(hT * hT, axis=1, keepdims=True).reshape(1, CH)


def _k4_body(h_ref, v_ref, pr0_ref, idx_ref, sp_ref, ssp_ref,
             gp_r, betap_r, wp1T_r, bp1_r,
             s1_ref, ss1_ref, gw1_r, bw1_r, ww1T_r, bw1l_r,
             out_ref):
    scale1, shift1 = _bn_scale_shift(s1_ref, ss1_ref, gw1_r[...], bw1_r[...])
    hnT = jnp.maximum(h_ref[0] * scale1.reshape(CH, 1) + shift1.reshape(CH, 1), 0.0)
    w1 = lax.dot_general(hnT, ww1T_r[...], (((0,), (0,)), ((), ())),
                         preferred_element_type=jnp.float32) + bw1l_r[...]
    w3 = w1.reshape(PC, NS, CH)
    mx = jnp.max(w3, axis=1, keepdims=True)
    e = jnp.exp(w3 - mx)
    rinv = 1.0 / jnp.sum(e, axis=1, keepdims=True)
    wt = jnp.concatenate([e] * S, axis=2)

    p_r = _pos_feat(pr0_ref, sp_ref, ssp_ref, gp_r, betap_r, wp1T_r, bp1_r)
    sel = _onehot_rows(idx_ref[0])
    g_v = jnp.dot(sel, v_ref[0], preferred_element_type=jnp.float32)
    a = (g_v + p_r).reshape(PC, NS, C)
    acc = jnp.sum(a * wt, axis=1)
    rt = jnp.concatenate([rinv[:, 0, :]] * S, axis=1)
    out_ref[0] = acc * rt


def kernel(p, x, Wq, bq, Wk, bk, Wv, bv, Wp0, bp0, gp, betap, Wp1, bp1,
           gw0, bw0, Ww0, bw0l, gw1, bw1, Ww1, bw1l):
    f32 = jnp.float32
    pT = jnp.transpose(p, (0, 2, 1))
    wqT, wkT, wvT = Wq.T, Wk.T, Wv.T
    wp0T, wp1T, ww0T, ww1T = Wp0.T, Wp1.T, Ww0.T, Ww1.T
    r2 = lambda a: a.reshape(1, -1)
    bq2, bk2, bv2, bp12, bw1l2 = map(r2, (bq, bk, bv, bp1, bw1l))
    bw0l2 = bw0l.reshape(-1, 1)
    bp02 = bp0.reshape(-1, 1)
    gp2, betap2, gw02, bw02, gw12, bw12 = map(r2, (gp, betap, gw0, bw0, gw1, bw1))

    grid = (B, NCHUNK)
    full = lambda shape: pl.BlockSpec(shape, lambda b, c: (0,) * len(shape))
    bc = lambda *shape: pl.BlockSpec((1,) + shape, lambda b, c: (b, c) + (0,) * (len(shape) - 1))
    bonly = lambda *shape: pl.BlockSpec((1,) + shape, lambda b, c: (b,) + (0,) * len(shape))
    stat = lambda d: pl.BlockSpec((1, 1, 1, d), lambda b, c: (b, c, 0, 0))
    sd = jax.ShapeDtypeStruct

    q, k, v, idx, pr0, sp, ssp = pl.pallas_call(
        _k1_body,
        grid=grid,
        in_specs=[bonly(C, N), bc(PC, 3), bonly(3, N),
                  full((C, C)), full((1, C)), full((C, C)), full((1, C)),
                  full((C, C)), full((1, C)), full((3, 3)), full((3, 1))],
        out_specs=[bc(PC, C), bc(PC, C), bc(PC, C), bc(PC, NS), bonly(3, N * NS),
                   stat(3), stat(3)],
        out_shape=[sd((B, N, C), f32), sd((B, N, C), f32), sd((B, N, C), f32),
                   sd((B, N, NS), f32), sd((B, 3, N * NS), f32),
                   sd((B, NCHUNK, 1, 3), f32), sd((B, NCHUNK, 1, 3), f32)],
    )(x, p, pT, wqT, bq2, wkT, bk2, wvT, bv2, wp0T, bp02)

    s0, ss0 = pl.pallas_call(
        _k2_body,
        grid=grid,
        in_specs=[bc(PC, C), bonly(N, C), bonly(3, N * NS), bc(PC, NS),
                  full((B, NCHUNK, 1, 3)), full((B, NCHUNK, 1, 3)),
                  full((1, 3)), full((1, 3)), full((3, C)), full((1, C))],
        out_specs=[stat(C), stat(C)],
        out_shape=[sd((B, NCHUNK, 1, C), f32), sd((B, NCHUNK, 1, C), f32)],
    )(q, k, pr0, idx, sp, ssp, gp2, betap2, wp1T, bp12)

    h, s1, ss1 = pl.pallas_call(
        _k3_body,
        grid=grid,
        in_specs=[bc(PC, C), bonly(N, C), bonly(3, N * NS), bc(PC, NS),
                  full((B, NCHUNK, 1, 3)), full((B, NCHUNK, 1, 3)),
                  full((1, 3)), full((1, 3)), full((3, C)), full((1, C)),
                  full((B, NCHUNK, 1, C)), full((B, NCHUNK, 1, C)),
                  full((1, C)), full((1, C)), full((C, CH)), full((CH, 1))],
        out_specs=[bonly(CH, N * NS), stat(CH), stat(CH)],
        out_shape=[sd((B, CH, N * NS), f32),
                   sd((B, NCHUNK, 1, CH), f32), sd((B, NCHUNK, 1, CH), f32)],
    )(q, k, pr0, idx, sp, ssp, gp2, betap2, wp1T, bp12,
      s0, ss0, gw02, bw02, ww0T, bw0l2)

    out = pl.pallas_call(
        _k4_body,
        grid=grid,
        in_specs=[bonly(CH, N * NS), bonly(N, C), bonly(3, N * NS), bc(PC, NS),
                  full((B, NCHUNK, 1, 3)), full((B, NCHUNK, 1, 3)),
                  full((1, 3)), full((1, 3)), full((3, C)), full((1, C)),
                  full((B, NCHUNK, 1, CH)), full((B, NCHUNK, 1, CH)),
                  full((1, CH)), full((1, CH)), full((CH, CH)), full((1, CH))],
        out_specs=[bc(PC, C)],
        out_shape=[sd((B, N, C), f32)],
    )(h, v, pr0, idx, sp, ssp, gp2, betap2, wp1T, bp12,
      s1, ss1, gw12, bw12, ww1T, bw1l2)[0]

    return out


# speedup vs baseline: 1.0126x; 1.0126x over previous
import jax
import jax.numpy as jnp
from jax import lax
from jax.experimental import pallas as pl

B, N, NS = 8, 512, 16
C = 256
S = 8
CH = C // S
NCHUNK = 1
PC = N // NCHUNK
RC = PC * NS
CNT = float(B * N * NS)
EPS = 1e-5


def _onehot_rows(idxc):
    tgt = lax.broadcasted_iota(jnp.int32, (PC, NS, N), 2).astype(jnp.float32)
    sel = jnp.where(idxc[:, :, None] == tgt, 1.0, 0.0)
    return sel.reshape(RC, N)


def _rep_rows(a):
    d = a.shape[-1]
    return jnp.broadcast_to(a[:, None, :], (PC, NS, d)).reshape(RC, d)


def _bn_scale_shift(s_ref, ss_ref, gamma, beta):
    d = s_ref.shape[-1]
    ssum = jnp.sum(s_ref[...].reshape(B * NCHUNK, d), axis=0, keepdims=True)
    sssum = jnp.sum(ss_ref[...].reshape(B * NCHUNK, d), axis=0, keepdims=True)
    mean = ssum / CNT
    var = sssum / CNT - mean * mean
    scale = gamma / jnp.sqrt(var + EPS)
    shift = beta - mean * scale
    return scale, shift


def _pos_feat(pr0_ref, sp_ref, ssp_ref, gp_r, betap_r, wp1T_r, bp1_r):
    scalep, shiftp = _bn_scale_shift(sp_ref, ssp_ref, gp_r[...], betap_r[...])
    prn = jnp.maximum(pr0_ref[0] * scalep.reshape(3, 1) + shiftp.reshape(3, 1), 0.0)
    return lax.dot_general(prn, wp1T_r[...], (((0,), (0,)), ((), ())),
                           preferred_element_type=jnp.float32) + bp1_r[...]


def _k1_body(xt_ref, p_ref, pT_ref, wqT, bq, wkT, bk, wvT, bv, wp0T, bp0,
             q_ref, k_ref, v_ref, idx_ref, pr0_ref, sp_ref, ssp_ref):
    xtc = xt_ref[0]
    q_ref[0] = jnp.dot(xtc, wqT[...], preferred_element_type=jnp.float32) + bq[...]
    k_ref[0] = jnp.dot(xtc, wkT[...], preferred_element_type=jnp.float32) + bk[...]
    v_ref[0] = jnp.dot(xtc, wvT[...], preferred_element_type=jnp.float32) + bv[...]

    pc = p_ref[0]
    pT = pT_ref[0]
    dx = pc[:, 0:1] - pT[0:1, :]
    dy = pc[:, 1:2] - pT[1:2, :]
    dz = pc[:, 2:3] - pT[2:3, :]
    d2 = (dx * dx + dy * dy) + dz * dz

    colid = lax.broadcasted_iota(jnp.int32, (PC, N), 1).astype(jnp.float32)
    work = d2
    cols = []
    for _ in range(NS):
        m = jnp.min(work, axis=1, keepdims=True)
        cand = jnp.where(work == m, colid, jnp.float32(N))
        am = jnp.min(cand, axis=1, keepdims=True)
        cols.append(am)
        work = jnp.where(colid == am, jnp.inf, work)
    idxc = jnp.concatenate(cols, axis=1)
    idx_ref[0] = idxc

    sel = _onehot_rows(idxc)
    gp3 = lax.dot_general(sel, pT, (((1,), (1,)), ((), ())),
                          preferred_element_type=jnp.float32)
    prel = gp3 - _rep_rows(pc)
    pr0T = lax.dot_general(wp0T[...], prel, (((0,), (1,)), ((), ())),
                           preferred_element_type=jnp.float32) + bp0[...]
    pr0_ref[0] = pr0T
    sp_ref[0, 0] = jnp.sum(pr0T, axis=1, keepdims=True).reshape(1, 3)
    ssp_ref[0, 0] = jnp.sum(pr0T * pr0T, axis=1, keepdims=True).reshape(1, 3)


def _w0(q_ref, k_ref, idx_ref, p_r):
    sel = _onehot_rows(idx_ref[0])
    g_k = jnp.dot(sel, k_ref[0], preferred_element_type=jnp.float32)
    return g_k - _rep_rows(q_ref[0]) + p_r


def _k2_body(q_ref, k_ref, pr0_ref, idx_ref, sp_ref, ssp_ref,
             gp_r, betap_r, wp1T_r, bp1_r, s0_ref, ss0_ref):
    p_r = _pos_feat(pr0_ref, sp_ref, ssp_ref, gp_r, betap_r, wp1T_r, bp1_r)
    w0 = _w0(q_ref, k_ref, idx_ref, p_r)
    s0_ref[0, 0] = jnp.sum(w0, axis=0, keepdims=True)
    ss0_ref[0, 0] = jnp.sum(w0 * w0, axis=0, keepdims=True)


def _k3_body(q_ref, k_ref, pr0_ref, idx_ref, sp_ref, ssp_ref,
             gp_r, betap_r, wp1T_r, bp1_r,
             s0_ref, ss0_ref, gw0_r, bw0_r, ww0T_r, bw0l_r,
             h_ref, s1_ref, ss1_ref):
    p_r = _pos_feat(pr0_ref, sp_ref, ssp_ref, gp_r, betap_r, wp1T_r, bp1_r)
    w0 = _w0(q_ref, k_ref, idx_ref, p_r)
    scale0, shift0 = _bn_scale_shift(s0_ref, ss0_ref, gw0_r[...], bw0_r[...])
    w0n = jnp.maximum(w0 * scale0 + shift0, 0.0)
    hT = lax.dot_general(ww0T_r[...], w0n, (((0,), (1,)), ((), ())),
                         preferred_element_type=jnp.float32) + bw0l_r[...]
    h_ref[0] = hT
    s1_ref[0, 0] = jnp.sum(hT, axis=1, keepdims=True).reshape(1, CH)
    ss1_ref[0, 0] = jnp.sum(hT * hT, axis=1, keepdims=True).reshape(1, CH)


def _k4_body(h_ref, v_ref, pr0_ref, idx_ref, sp_ref, ssp_ref,
             gp_r, betap_r, wp1T_r, bp1_r,
             s1_ref, ss1_ref, gw1_r, bw1_r, ww1T_r, bw1l_r,
             out_ref):
    scale1, shift1 = _bn_scale_shift(s1_ref, ss1_ref, gw1_r[...], bw1_r[...])
    hnT = jnp.maximum(h_ref[0] * scale1.reshape(CH, 1) + shift1.reshape(CH, 1), 0.0)
    w1 = lax.dot_general(hnT, ww1T_r[...], (((0,), (0,)), ((), ())),
                         preferred_element_type=jnp.float32) + bw1l_r[...]
    w3 = w1.reshape(PC, NS, CH)
    e = jnp.exp(w3)
    rinv = 1.0 / jnp.sum(e, axis=1, keepdims=True)
    wt = jnp.concatenate([e] * S, axis=2)

    p_r = _pos_feat(pr0_ref, sp_ref, ssp_ref, gp_r, betap_r, wp1T_r, bp1_r)
    sel = _onehot_rows(idx_ref[0])
    g_v = jnp.dot(sel, v_ref[0], preferred_element_type=jnp.float32)
    a = (g_v + p_r).reshape(PC, NS, C)
    acc = jnp.sum(a * wt, axis=1)
    rt = jnp.concatenate([rinv[:, 0, :]] * S, axis=1)
    out_ref[0] = acc * rt


def kernel(p, x, Wq, bq, Wk, bk, Wv, bv, Wp0, bp0, gp, betap, Wp1, bp1,
           gw0, bw0, Ww0, bw0l, gw1, bw1, Ww1, bw1l):
    f32 = jnp.float32
    xt = jnp.transpose(x, (0, 2, 1))
    pT = jnp.transpose(p, (0, 2, 1))
    wqT, wkT, wvT = Wq.T, Wk.T, Wv.T
    wp0T, wp1T, ww0T, ww1T = Wp0.T, Wp1.T, Ww0.T, Ww1.T
    r2 = lambda a: a.reshape(1, -1)
    bq2, bk2, bv2, bp12, bw1l2 = map(r2, (bq, bk, bv, bp1, bw1l))
    bw0l2 = bw0l.reshape(-1, 1)
    bp02 = bp0.reshape(-1, 1)
    gp2, betap2, gw02, bw02, gw12, bw12 = map(r2, (gp, betap, gw0, bw0, gw1, bw1))

    grid = (B, NCHUNK)
    full = lambda shape: pl.BlockSpec(shape, lambda b, c: (0,) * len(shape))
    bc = lambda *shape: pl.BlockSpec((1,) + shape, lambda b, c: (b, c) + (0,) * (len(shape) - 1))
    bonly = lambda *shape: pl.BlockSpec((1,) + shape, lambda b, c: (b,) + (0,) * len(shape))
    stat = lambda d: pl.BlockSpec((1, 1, 1, d), lambda b, c: (b, c, 0, 0))
    sd = jax.ShapeDtypeStruct

    q, k, v, idx, pr0, sp, ssp = pl.pallas_call(
        _k1_body,
        grid=grid,
        in_specs=[bc(PC, C), bc(PC, 3), bonly(3, N),
                  full((C, C)), full((1, C)), full((C, C)), full((1, C)),
                  full((C, C)), full((1, C)), full((3, 3)), full((3, 1))],
        out_specs=[bc(PC, C), bc(PC, C), bc(PC, C), bc(PC, NS), bonly(3, N * NS),
                   stat(3), stat(3)],
        out_shape=[sd((B, N, C), f32), sd((B, N, C), f32), sd((B, N, C), f32),
                   sd((B, N, NS), f32), sd((B, 3, N * NS), f32),
                   sd((B, NCHUNK, 1, 3), f32), sd((B, NCHUNK, 1, 3), f32)],
    )(xt, p, pT, wqT, bq2, wkT, bk2, wvT, bv2, wp0T, bp02)

    s0, ss0 = pl.pallas_call(
        _k2_body,
        grid=grid,
        in_specs=[bc(PC, C), bonly(N, C), bonly(3, N * NS), bc(PC, NS),
                  full((B, NCHUNK, 1, 3)), full((B, NCHUNK, 1, 3)),
                  full((1, 3)), full((1, 3)), full((3, C)), full((1, C))],
        out_specs=[stat(C), stat(C)],
        out_shape=[sd((B, NCHUNK, 1, C), f32), sd((B, NCHUNK, 1, C), f32)],
    )(q, k, pr0, idx, sp, ssp, gp2, betap2, wp1T, bp12)

    h, s1, ss1 = pl.pallas_call(
        _k3_body,
        grid=grid,
        in_specs=[bc(PC, C), bonly(N, C), bonly(3, N * NS), bc(PC, NS),
                  full((B, NCHUNK, 1, 3)), full((B, NCHUNK, 1, 3)),
                  full((1, 3)), full((1, 3)), full((3, C)), full((1, C)),
                  full((B, NCHUNK, 1, C)), full((B, NCHUNK, 1, C)),
                  full((1, C)), full((1, C)), full((C, CH)), full((CH, 1))],
        out_specs=[bonly(CH, N * NS), stat(CH), stat(CH)],
        out_shape=[sd((B, CH, N * NS), f32),
                   sd((B, NCHUNK, 1, CH), f32), sd((B, NCHUNK, 1, CH), f32)],
    )(q, k, pr0, idx, sp, ssp, gp2, betap2, wp1T, bp12,
      s0, ss0, gw02, bw02, ww0T, bw0l2)

    out = pl.pallas_call(
        _k4_body,
        grid=grid,
        in_specs=[bonly(CH, N * NS), bonly(N, C), bonly(3, N * NS), bc(PC, NS),
                  full((B, NCHUNK, 1, 3)), full((B, NCHUNK, 1, 3)),
                  full((1, 3)), full((1, 3)), full((3, C)), full((1, C)),
                  full((B, NCHUNK, 1, CH)), full((B, NCHUNK, 1, CH)),
                  full((1, CH)), full((1, CH)), full((CH, CH)), full((1, CH))],
        out_specs=[bc(PC, C)],
        out_shape=[sd((B, N, C), f32)],
    )(h, v, pr0, idx, sp, ssp, gp2, betap2, wp1T, bp12,
      s1, ss1, gw12, bw12, ww1T, bw1l2)[0]

    return out


# speedup vs baseline: 1.0167x; 1.0041x over previous
import jax
import jax.numpy as jnp
from jax import lax
from jax.experimental import pallas as pl

B, N, NS = 8, 512, 16
C = 256
S = 8
CH = C // S
NCHUNK = 1
PC = N // NCHUNK
RC = PC * NS
CNT = float(B * N * NS)
EPS = 1e-5


def _onehot_rows(idxc):
    tgt = lax.broadcasted_iota(jnp.int32, (PC, NS, N), 2).astype(jnp.float32)
    sel = jnp.where(idxc[:, :, None] == tgt, 1.0, 0.0)
    return sel.reshape(RC, N)


def _rep_rows(a):
    d = a.shape[-1]
    return jnp.broadcast_to(a[:, None, :], (PC, NS, d)).reshape(RC, d)


def _bn_scale_shift(s_ref, ss_ref, gamma, beta):
    d = s_ref.shape[-1]
    ssum = jnp.sum(s_ref[...].reshape(B * NCHUNK, d), axis=0, keepdims=True)
    sssum = jnp.sum(ss_ref[...].reshape(B * NCHUNK, d), axis=0, keepdims=True)
    mean = ssum / CNT
    var = sssum / CNT - mean * mean
    scale = gamma / jnp.sqrt(var + EPS)
    shift = beta - mean * scale
    return scale, shift


def _pos_feat(pr0_ref, sp_ref, ssp_ref, gp_r, betap_r, wp1T_r, bp1_r):
    scalep, shiftp = _bn_scale_shift(sp_ref, ssp_ref, gp_r[...], betap_r[...])
    prn = jnp.maximum(pr0_ref[0] * scalep.reshape(3, 1) + shiftp.reshape(3, 1), 0.0)
    return lax.dot_general(prn, wp1T_r[...], (((0,), (0,)), ((), ())),
                           preferred_element_type=jnp.float32) + bp1_r[...]


def _k1_body(xt_ref, p_ref, pT_ref, wqT, bq, wkT, bk, wvT, bv, wp0T, bp0,
             q_ref, k_ref, v_ref, idx_ref, pr0_ref, sp_ref, ssp_ref):
    xtc = xt_ref[0]
    q_ref[0] = jnp.dot(xtc, wqT[...], preferred_element_type=jnp.float32) + bq[...]
    k_ref[0] = jnp.dot(xtc, wkT[...], preferred_element_type=jnp.float32) + bk[...]
    v_ref[0] = jnp.dot(xtc, wvT[...], preferred_element_type=jnp.float32) + bv[...]

    pc = p_ref[0]
    pT = pT_ref[0]
    dx = pc[:, 0:1] - pT[0:1, :]
    dy = pc[:, 1:2] - pT[1:2, :]
    dz = pc[:, 2:3] - pT[2:3, :]
    d2 = (dx * dx + dy * dy) + dz * dz

    colid = lax.broadcasted_iota(jnp.int32, (PC, N), 1).astype(jnp.float32)
    work = d2
    cols = []
    for _ in range(NS):
        m = jnp.min(work, axis=1, keepdims=True)
        cand = jnp.where(work == m, colid, jnp.float32(N))
        am = jnp.min(cand, axis=1, keepdims=True)
        cols.append(am)
        work = jnp.where(colid == am, jnp.inf, work)
    idxc = jnp.concatenate(cols, axis=1)
    idx_ref[0] = idxc

    sel = _onehot_rows(idxc)
    gp3 = jnp.dot(sel, pc, preferred_element_type=jnp.float32)
    prel = gp3 - _rep_rows(pc)
    pr0T = lax.dot_general(wp0T[...], prel, (((0,), (1,)), ((), ())),
                           preferred_element_type=jnp.float32) + bp0[...]
    pr0_ref[0] = pr0T
    sp_ref[0, 0] = jnp.sum(pr0T, axis=1, keepdims=True).reshape(1, 3)
    ssp_ref[0, 0] = jnp.sum(pr0T * pr0T, axis=1, keepdims=True).reshape(1, 3)


def _w0(q_ref, k_ref, idx_ref, p_r):
    sel = _onehot_rows(idx_ref[0])
    g_k = jnp.dot(sel, k_ref[0], preferred_element_type=jnp.float32)
    return g_k - _rep_rows(q_ref[0]) + p_r


def _k2_body(q_ref, k_ref, pr0_ref, idx_ref, sp_ref, ssp_ref,
             gp_r, betap_r, wp1T_r, bp1_r, s0_ref, ss0_ref):
    p_r = _pos_feat(pr0_ref, sp_ref, ssp_ref, gp_r, betap_r, wp1T_r, bp1_r)
    w0 = _w0(q_ref, k_ref, idx_ref, p_r)
    s0_ref[0, 0] = jnp.sum(w0, axis=0, keepdims=True)
    ss0_ref[0, 0] = jnp.sum(w0 * w0, axis=0, keepdims=True)


def _k3_body(q_ref, k_ref, pr0_ref, idx_ref, sp_ref, ssp_ref,
             gp_r, betap_r, wp1T_r, bp1_r,
             s0_ref, ss0_ref, gw0_r, bw0_r, ww0T_r, bw0l_r,
             h_ref, s1_ref, ss1_ref):
    p_r = _pos_feat(pr0_ref, sp_ref, ssp_ref, gp_r, betap_r, wp1T_r, bp1_r)
    w0 = _w0(q_ref, k_ref, idx_ref, p_r)
    scale0, shift0 = _bn_scale_shift(s0_ref, ss0_ref, gw0_r[...], bw0_r[...])
    w0n = jnp.maximum(w0 * scale0 + shift0, 0.0)
    hT = lax.dot_general(ww0T_r[...], w0n, (((0,), (1,)), ((), ())),
                         preferred_element_type=jnp.float32) + bw0l_r[...]
    h_ref[0] = hT
    s1_ref[0, 0] = jnp.sum(hT, axis=1, keepdims=True).reshape(1, CH)
    ss1_ref[0, 0] = jnp.sum(hT * hT, axis=1, keepdims=True).reshape(1, CH)


def _k4_body(h_ref, v_ref, pr0_ref, idx_ref, sp_ref, ssp_ref,
             gp_r, betap_r, wp1T_r, bp1_r,
             s1_ref, ss1_ref, gw1_r, bw1_r, ww1T_r, bw1l_r,
             out_ref):
    scale1, shift1 = _bn_scale_shift(s1_ref, ss1_ref, gw1_r[...], bw1_r[...])
    hnT = jnp.maximum(h_ref[0] * scale1.reshape(CH, 1) + shift1.reshape(CH, 1), 0.0)
    w1 = lax.dot_general(hnT, ww1T_r[...], (((0,), (0,)), ((), ())),
                         preferred_element_type=jnp.float32) + bw1l_r[...]
    w3 = w1.reshape(PC, NS, CH)
    mx = jnp.max(w3, axis=1, keepdims=True)
    e = jnp.exp(w3 - mx)
    rinv = 1.0 / jnp.sum(e, axis=1, keepdims=True)
    wt = jnp.concatenate([e] * S, axis=2)

    p_r = _pos_feat(pr0_ref, sp_ref, ssp_ref, gp_r, betap_r, wp1T_r, bp1_r)
    sel = _onehot_rows(idx_ref[0])
    g_v = jnp.dot(sel, v_ref[0], preferred_element_type=jnp.float32)
    a = (g_v + p_r).reshape(PC, NS, C)
    acc = jnp.sum(a * wt, axis=1)
    rt = jnp.concatenate([rinv[:, 0, :]] * S, axis=1)
    out_ref[0] = acc * rt


def kernel(p, x, Wq, bq, Wk, bk, Wv, bv, Wp0, bp0, gp, betap, Wp1, bp1,
           gw0, bw0, Ww0, bw0l, gw1, bw1, Ww1, bw1l):
    f32 = jnp.float32
    xt = jnp.transpose(x, (0, 2, 1))
    pT = jnp.transpose(p, (0, 2, 1))
    wqT, wkT, wvT = Wq.T, Wk.T, Wv.T
    wp0T, wp1T, ww0T, ww1T = Wp0.T, Wp1.T, Ww0.T, Ww1.T
    r2 = lambda a: a.reshape(1, -1)
    bq2, bk2, bv2, bp12, bw1l2 = map(r2, (bq, bk, bv, bp1, bw1l))
    bw0l2 = bw0l.reshape(-1, 1)
    bp02 = bp0.reshape(-1, 1)
    gp2, betap2, gw02, bw02, gw12, bw12 = map(r2, (gp, betap, gw0, bw0, gw1, bw1))

    grid = (B, NCHUNK)
    full = lambda shape: pl.BlockSpec(shape, lambda b, c: (0,) * len(shape))
    bc = lambda *shape: pl.BlockSpec((1,) + shape, lambda b, c: (b, c) + (0,) * (len(shape) - 1))
    bonly = lambda *shape: pl.BlockSpec((1,) + shape, lambda b, c: (b,) + (0,) * len(shape))
    stat = lambda d: pl.BlockSpec((1, 1, 1, d), lambda b, c: (b, c, 0, 0))
    sd = jax.ShapeDtypeStruct

    q, k, v, idx, pr0, sp, ssp = pl.pallas_call(
        _k1_body,
        grid=grid,
        in_specs=[bc(PC, C), bc(PC, 3), bonly(3, N),
                  full((C, C)), full((1, C)), full((C, C)), full((1, C)),
                  full((C, C)), full((1, C)), full((3, 3)), full((3, 1))],
        out_specs=[bc(PC, C), bc(PC, C), bc(PC, C), bc(PC, NS), bonly(3, N * NS),
                   stat(3), stat(3)],
        out_shape=[sd((B, N, C), f32), sd((B, N, C), f32), sd((B, N, C), f32),
                   sd((B, N, NS), f32), sd((B, 3, N * NS), f32),
                   sd((B, NCHUNK, 1, 3), f32), sd((B, NCHUNK, 1, 3), f32)],
    )(xt, p, pT, wqT, bq2, wkT, bk2, wvT, bv2, wp0T, bp02)

    s0, ss0 = pl.pallas_call(
        _k2_body,
        grid=grid,
        in_specs=[bc(PC, C), bonly(N, C), bonly(3, N * NS), bc(PC, NS),
                  full((B, NCHUNK, 1, 3)), full((B, NCHUNK, 1, 3)),
                  full((1, 3)), full((1, 3)), full((3, C)), full((1, C))],
        out_specs=[stat(C), stat(C)],
        out_shape=[sd((B, NCHUNK, 1, C), f32), sd((B, NCHUNK, 1, C), f32)],
    )(q, k, pr0, idx, sp, ssp, gp2, betap2, wp1T, bp12)

    h, s1, ss1 = pl.pallas_call(
        _k3_body,
        grid=grid,
        in_specs=[bc(PC, C), bonly(N, C), bonly(3, N * NS), bc(PC, NS),
                  full((B, NCHUNK, 1, 3)), full((B, NCHUNK, 1, 3)),
                  full((1, 3)), full((1, 3)), full((3, C)), full((1, C)),
                  full((B, NCHUNK, 1, C)), full((B, NCHUNK, 1, C)),
                  full((1, C)), full((1, C)), full((C, CH)), full((CH, 1))],
        out_specs=[bonly(CH, N * NS), stat(CH), stat(CH)],
        out_shape=[sd((B, CH, N * NS), f32),
                   sd((B, NCHUNK, 1, CH), f32), sd((B, NCHUNK, 1, CH), f32)],
    )(q, k, pr0, idx, sp, ssp, gp2, betap2, wp1T, bp12,
      s0, ss0, gw02, bw02, ww0T, bw0l2)

    out = pl.pallas_call(
        _k4_body,
        grid=grid,
        in_specs=[bonly(CH, N * NS), bonly(N, C), bonly(3, N * NS), bc(PC, NS),
                  full((B, NCHUNK, 1, 3)), full((B, NCHUNK, 1, 3)),
                  full((1, 3)), full((1, 3)), full((3, C)), full((1, C)),
                  full((B, NCHUNK, 1, CH)), full((B, NCHUNK, 1, CH)),
                  full((1, CH)), full((1, CH)), full((CH, CH)), full((1, CH))],
        out_specs=[bc(PC, C)],
        out_shape=[sd((B, N, C), f32)],
    )(h, v, pr0, idx, sp, ssp, gp2, betap2, wp1T, bp12,
      s1, ss1, gw12, bw12, ww1T, bw1l2)[0]

    return out
